# baseline (speedup 1.0000x reference)
import functools

import jax
import jax.numpy as jnp
from jax.experimental import pallas as pl
from jax.experimental.pallas import tpu as pltpu


def _conv_bn_silu_kernel(x_ref, w_ref, s_ref, o_ref, *, h, w, b_tile):
    npix = h * w
    wk = w_ref[...]
    shift = s_ref[...]

    p = jax.lax.broadcasted_iota(jnp.int32, (1, npix), 1)
    wi = p % w
    hi = p // w

    base = pl.program_id(0) * b_tile
    for b in range(b_tile):
        x = x_ref[base + b].astype(jnp.bfloat16)
        cols = []
        for i in range(3):
            for j in range(3):
                d = (i - 1) * w + (j - 1)
                if d != 0:
                    sh = jnp.concatenate([x[:, d:], x[:, :d]], axis=1)
                else:
                    sh = x
                valid = ((wi + j >= 1) & (wi + j <= w)
                         & (hi + i >= 1) & (hi + i <= h))
                cols.append(jnp.where(valid, sh, jnp.bfloat16(0)))
        col = jnp.concatenate(cols, axis=0)
        y = jnp.dot(wk, col, preferred_element_type=jnp.float32)
        y = y + shift
        y = y * pl.reciprocal(1.0 + jnp.exp(-y), approx=True)
        o_ref[b] = y


def kernel(x_nchw, conv_weight, bn_weight, bn_bias,
           bn_running_mean, bn_running_var):
    eps = 1e-5
    n, cin, h, w = x_nchw.shape
    cout = conv_weight.shape[0]
    npix = h * w

    scale = bn_weight / jnp.sqrt(bn_running_var + eps)
    shift = bn_bias - bn_running_mean * scale
    w_folded = conv_weight * scale[:, None, None, None]
    w_k = jnp.transpose(w_folded, (0, 2, 3, 1)).reshape(
        cout, 9 * cin).astype(jnp.bfloat16)

    x = x_nchw.reshape(n, cin, npix)

    b_tile = 8
    g = pl.cdiv(n, b_tile)
    n_pad = g * b_tile
    if n_pad != n:
        x = jnp.pad(x, ((0, n_pad - n), (0, 0), (0, 0)))

    out = pl.pallas_call(
        functools.partial(_conv_bn_silu_kernel, h=h, w=w, b_tile=b_tile),
        out_shape=jax.ShapeDtypeStruct((n_pad, cout, npix), jnp.float32),
        grid=(g,),
        in_specs=[
            pl.BlockSpec((n_pad, cin, npix), lambda b: (0, 0, 0)),
            pl.BlockSpec((cout, 9 * cin), lambda b: (0, 0)),
            pl.BlockSpec((cout, 1), lambda b: (0, 0)),
        ],
        out_specs=pl.BlockSpec((b_tile, cout, npix), lambda b: (b, 0, 0)),
        compiler_params=pltpu.CompilerParams(
            dimension_semantics=("arbitrary",),
            vmem_limit_bytes=56 * 1024 * 1024),
    )(x, w_k, shift.reshape(cout, 1))

    return out[:n].reshape(n, cout, h, w)


# speedup vs baseline: 1.0765x; 1.0765x over previous
import functools

import jax
import jax.numpy as jnp
from jax.experimental import pallas as pl
from jax.experimental.pallas import tpu as pltpu


def _conv_bn_silu_kernel(x_ref, w_ref, s_ref, o_ref, *, h, w, b_tile):
    npix = h * w
    wk = w_ref[...]
    shift = s_ref[...]

    wi = jax.lax.broadcasted_iota(jnp.int32, (1, npix), 1) % w

    cin = x_ref.shape[1]
    for b in range(b_tile):
        x = x_ref[b].astype(jnp.bfloat16)
        cols = []
        for i in range(3):
            for j in range(3):
                d = (i - 1) * w + (j - 1)
                if d > 0:
                    sh = jnp.concatenate(
                        [x[:, d:], jnp.zeros((cin, d), jnp.bfloat16)], axis=1)
                elif d < 0:
                    sh = jnp.concatenate(
                        [jnp.zeros((cin, -d), jnp.bfloat16), x[:, :d]], axis=1)
                else:
                    sh = x
                if j != 1:
                    valid = (wi + j >= 1) & (wi + j <= w)
                    sh = jnp.where(valid, sh, jnp.bfloat16(0))
                cols.append(sh)
        col = jnp.concatenate(cols, axis=0)
        y = jnp.dot(wk, col, preferred_element_type=jnp.float32)
        y = y + shift
        y = y * pl.reciprocal(1.0 + jnp.exp(-y), approx=True)
        o_ref[b] = y


def kernel(x_nchw, conv_weight, bn_weight, bn_bias,
           bn_running_mean, bn_running_var):
    eps = 1e-5
    n, cin, h, w = x_nchw.shape
    cout = conv_weight.shape[0]
    npix = h * w

    scale = bn_weight / jnp.sqrt(bn_running_var + eps)
    shift = bn_bias - bn_running_mean * scale
    w_folded = conv_weight * scale[:, None, None, None]
    w_k = jnp.transpose(w_folded, (0, 2, 3, 1)).reshape(
        cout, 9 * cin).astype(jnp.bfloat16)

    x = x_nchw.reshape(n, cin, npix)

    b_tile = 8
    g = pl.cdiv(n, b_tile)
    n_pad = g * b_tile
    if n_pad != n:
        x = jnp.pad(x, ((0, n_pad - n), (0, 0), (0, 0)))

    out = pl.pallas_call(
        functools.partial(_conv_bn_silu_kernel, h=h, w=w, b_tile=b_tile),
        out_shape=jax.ShapeDtypeStruct((n_pad, cout, npix), jnp.float32),
        grid=(g,),
        in_specs=[
            pl.BlockSpec((b_tile, cin, npix), lambda b: (b, 0, 0)),
            pl.BlockSpec((cout, 9 * cin), lambda b: (0, 0)),
            pl.BlockSpec((cout, 1), lambda b: (0, 0)),
        ],
        out_specs=pl.BlockSpec((b_tile, cout, npix), lambda b: (b, 0, 0)),
        compiler_params=pltpu.CompilerParams(
            dimension_semantics=("parallel",),
            vmem_limit_bytes=32 * 1024 * 1024),
    )(x, w_k, shift.reshape(cout, 1))

    return out[:n].reshape(n, cout, h, w)


# speedup vs baseline: 1.0815x; 1.0047x over previous
import functools

import jax
import jax.numpy as jnp
from jax.experimental import pallas as pl
from jax.experimental.pallas import tpu as pltpu


def _conv_bn_silu_kernel(x_ref, w_ref, s_ref, o_ref, *, h, w, b_tile):
    npix = h * w
    wk = w_ref[...]
    shift = s_ref[...]

    wi = jax.lax.broadcasted_iota(jnp.int32, (1, npix), 1) % w

    cin = x_ref.shape[1]
    for b in range(b_tile):
        x = x_ref[b].astype(jnp.bfloat16)
        cols = []
        for i in range(3):
            for j in range(3):
                d = (i - 1) * w + (j - 1)
                if d > 0:
                    sh = jnp.concatenate(
                        [x[:, d:], jnp.zeros((cin, d), jnp.bfloat16)], axis=1)
                elif d < 0:
                    sh = jnp.concatenate(
                        [jnp.zeros((cin, -d), jnp.bfloat16), x[:, :d]], axis=1)
                else:
                    sh = x
                if j != 1:
                    valid = (wi + j >= 1) & (wi + j <= w)
                    sh = jnp.where(valid, sh, jnp.bfloat16(0))
                cols.append(sh)
        col = jnp.concatenate(cols, axis=0)
        y = jnp.dot(wk, col, preferred_element_type=jnp.float32)
        y = y + shift
        y = y * pl.reciprocal(1.0 + jnp.exp(-y), approx=True)
        o_ref[b] = y


def kernel(x_nchw, conv_weight, bn_weight, bn_bias,
           bn_running_mean, bn_running_var):
    eps = 1e-5
    n, cin, h, w = x_nchw.shape
    cout = conv_weight.shape[0]
    npix = h * w

    scale = bn_weight / jnp.sqrt(bn_running_var + eps)
    shift = bn_bias - bn_running_mean * scale
    w_folded = conv_weight * scale[:, None, None, None]
    w_k = jnp.transpose(w_folded, (0, 2, 3, 1)).reshape(
        cout, 9 * cin).astype(jnp.bfloat16)

    x = x_nchw.reshape(n, cin, npix)

    b_tile = 16
    g = pl.cdiv(n, b_tile)
    n_pad = g * b_tile
    if n_pad != n:
        x = jnp.pad(x, ((0, n_pad - n), (0, 0), (0, 0)))

    out = pl.pallas_call(
        functools.partial(_conv_bn_silu_kernel, h=h, w=w, b_tile=b_tile),
        out_shape=jax.ShapeDtypeStruct((n_pad, cout, npix), jnp.float32),
        grid=(g,),
        in_specs=[
            pl.BlockSpec((b_tile, cin, npix), lambda b: (b, 0, 0)),
            pl.BlockSpec((cout, 9 * cin), lambda b: (0, 0)),
            pl.BlockSpec((cout, 1), lambda b: (0, 0)),
        ],
        out_specs=pl.BlockSpec((b_tile, cout, npix), lambda b: (b, 0, 0)),
        compiler_params=pltpu.CompilerParams(
            dimension_semantics=("parallel",),
            vmem_limit_bytes=32 * 1024 * 1024),
    )(x, w_k, shift.reshape(cout, 1))

    return out[:n].reshape(n, cout, h, w)


# speedup vs baseline: 1.1072x; 1.0237x over previous
import functools

import jax
import jax.numpy as jnp
from jax.experimental import pallas as pl
from jax.experimental.pallas import tpu as pltpu


def _conv_bn_silu_kernel(x_ref, w_ref, s_ref, o_ref, *, h, w, b_tile):
    npix = h * w
    wk = w_ref[...]
    shift = s_ref[...]

    wi = jax.lax.broadcasted_iota(jnp.int32, (1, npix), 1) % w

    cin = x_ref.shape[1]
    for b in range(b_tile):
        x = x_ref[b].astype(jnp.bfloat16)
        cols = []
        for i in range(3):
            for j in range(3):
                d = (i - 1) * w + (j - 1)
                if d > 0:
                    sh = jnp.concatenate(
                        [x[:, d:], jnp.zeros((cin, d), jnp.bfloat16)], axis=1)
                elif d < 0:
                    sh = jnp.concatenate(
                        [jnp.zeros((cin, -d), jnp.bfloat16), x[:, :d]], axis=1)
                else:
                    sh = x
                if j != 1:
                    valid = (wi + j >= 1) & (wi + j <= w)
                    sh = jnp.where(valid, sh, jnp.bfloat16(0))
                cols.append(sh)
        col = jnp.concatenate(cols, axis=0)
        y = jnp.dot(wk, col, preferred_element_type=jnp.float32)
        y = y + shift
        y = 0.5 * y * (1.0 + jnp.tanh(0.5 * y))
        o_ref[b] = y


def kernel(x_nchw, conv_weight, bn_weight, bn_bias,
           bn_running_mean, bn_running_var):
    eps = 1e-5
    n, cin, h, w = x_nchw.shape
    cout = conv_weight.shape[0]
    npix = h * w

    scale = bn_weight / jnp.sqrt(bn_running_var + eps)
    shift = bn_bias - bn_running_mean * scale
    w_folded = conv_weight * scale[:, None, None, None]
    w_k = jnp.transpose(w_folded, (0, 2, 3, 1)).reshape(
        cout, 9 * cin).astype(jnp.bfloat16)

    x = x_nchw.reshape(n, cin, npix)

    b_tile = 16
    g = pl.cdiv(n, b_tile)
    n_pad = g * b_tile
    if n_pad != n:
        x = jnp.pad(x, ((0, n_pad - n), (0, 0), (0, 0)))

    out = pl.pallas_call(
        functools.partial(_conv_bn_silu_kernel, h=h, w=w, b_tile=b_tile),
        out_shape=jax.ShapeDtypeStruct((n_pad, cout, npix), jnp.float32),
        grid=(g,),
        in_specs=[
            pl.BlockSpec((b_tile, cin, npix), lambda b: (b, 0, 0)),
            pl.BlockSpec((cout, 9 * cin), lambda b: (0, 0)),
            pl.BlockSpec((cout, 1), lambda b: (0, 0)),
        ],
        out_specs=pl.BlockSpec((b_tile, cout, npix), lambda b: (b, 0, 0)),
        compiler_params=pltpu.CompilerParams(
            dimension_semantics=("parallel",),
            vmem_limit_bytes=32 * 1024 * 1024),
    )(x, w_k, shift.reshape(cout, 1))

    return out[:n].reshape(n, cout, h, w)


# speedup vs baseline: 1.1243x; 1.0155x over previous
import functools

import jax
import jax.numpy as jnp
from jax.experimental import pallas as pl
from jax.experimental.pallas import tpu as pltpu


def _conv_bn_silu_kernel(x_ref, w_ref, o_ref, *, h, w, b_tile):
    npix = h * w
    wk = w_ref[...]

    wi = jax.lax.broadcasted_iota(jnp.int32, (1, npix), 1) % w

    cin = x_ref.shape[1]
    for b in range(b_tile):
        x = x_ref[b].astype(jnp.bfloat16)
        cols = []
        for i in range(3):
            for j in range(3):
                d = (i - 1) * w + (j - 1)
                if d > 0:
                    sh = jnp.concatenate(
                        [x[:, d:], jnp.zeros((cin, d), jnp.bfloat16)], axis=1)
                elif d < 0:
                    sh = jnp.concatenate(
                        [jnp.zeros((cin, -d), jnp.bfloat16), x[:, :d]], axis=1)
                else:
                    sh = x
                if j != 1:
                    valid = (wi + j >= 1) & (wi + j <= w)
                    sh = jnp.where(valid, sh, jnp.bfloat16(0))
                cols.append(sh)
        cols.append(jnp.ones((8, npix), jnp.bfloat16))
        col = jnp.concatenate(cols, axis=0)
        y = jnp.dot(wk, col, preferred_element_type=jnp.float32)
        y = 0.5 * y * (1.0 + jnp.tanh(0.5 * y))
        o_ref[b] = y


def kernel(x_nchw, conv_weight, bn_weight, bn_bias,
           bn_running_mean, bn_running_var):
    eps = 1e-5
    n, cin, h, w = x_nchw.shape
    cout = conv_weight.shape[0]
    npix = h * w

    scale = bn_weight / jnp.sqrt(bn_running_var + eps)
    shift = bn_bias - bn_running_mean * scale
    w_folded = conv_weight * scale[:, None, None, None]
    w_k = jnp.transpose(w_folded, (0, 2, 3, 1)).reshape(cout, 9 * cin)
    w_k = jnp.concatenate(
        [w_k, shift[:, None], jnp.zeros((cout, 7), jnp.float32)],
        axis=1).astype(jnp.bfloat16)

    x = x_nchw.reshape(n, cin, npix)

    b_tile = 16
    g = pl.cdiv(n, b_tile)
    n_pad = g * b_tile
    if n_pad != n:
        x = jnp.pad(x, ((0, n_pad - n), (0, 0), (0, 0)))

    out = pl.pallas_call(
        functools.partial(_conv_bn_silu_kernel, h=h, w=w, b_tile=b_tile),
        out_shape=jax.ShapeDtypeStruct((n_pad, cout, npix), jnp.float32),
        grid=(g,),
        in_specs=[
            pl.BlockSpec((b_tile, cin, npix), lambda b: (b, 0, 0)),
            pl.BlockSpec((cout, 9 * cin + 8), lambda b: (0, 0)),
        ],
        out_specs=pl.BlockSpec((b_tile, cout, npix), lambda b: (b, 0, 0)),
        compiler_params=pltpu.CompilerParams(
            dimension_semantics=("parallel",),
            vmem_limit_bytes=32 * 1024 * 1024),
    )(x, w_k)

    return out[:n].reshape(n, cout, h, w)
